# baseline (speedup 1.0000x reference)
import functools

import jax
import jax.numpy as jnp
from jax import lax
from jax.experimental import pallas as pl
from jax.experimental.pallas import tpu as pltpu
from jax.experimental.pallas import tpu_sc as plsc

NN = 10000
NP = 10240
EE = 160000
EP = 163840
NC = 2
NS = 16
NW = NC * NS
EW = EP // NW
CH = 64
NCH = EW // CH
RPT = NP // NS
D_HID = 128
NBUF = 3


def _sc_mesh():
    return plsc.VectorSubcoreMesh(core_axis_name="c", subcore_axis_name="s",
                                  num_cores=NC, num_subcores=NS)


DCH = 128
DNCH = EW // DCH


@functools.partial(
    pl.kernel,
    out_type=jax.ShapeDtypeStruct((NC, NP), jnp.float32),
    mesh=_sc_mesh(),
    scratch_types=[
        pltpu.VMEM((DNCH, DCH), jnp.int32),
        pltpu.VMEM((DCH,), jnp.float32),
        pltpu.VMEM((RPT,), jnp.float32),
        pltpu.VMEM_SHARED((NP,), jnp.float32),
    ],
)
def _deg_kernel(dstm_hbm, degp_hbm, dst_v, ones_v, zz_v, deg_sh):
    cid = lax.axis_index("c")
    sid = lax.axis_index("s")
    wid = sid * NC + cid

    @pl.loop(0, DCH // 16)
    def _fill_ones(i):
        ones_v[pl.ds(i * 16, 16)] = jnp.ones((16,), jnp.float32)

    @pl.loop(0, RPT // 16)
    def _fill_zero(i):
        zz_v[pl.ds(i * 16, 16)] = jnp.zeros((16,), jnp.float32)

    pltpu.sync_copy(dstm_hbm.at[wid], dst_v)
    pltpu.sync_copy(zz_v, deg_sh.at[pl.ds(sid * RPT, RPT)])
    plsc.subcore_barrier()

    @pl.loop(0, DNCH)
    def _scatter(j):
        pltpu.sync_copy(ones_v, deg_sh.at[dst_v.at[j]], add=True)

    plsc.subcore_barrier()
    pltpu.sync_copy(deg_sh.at[pl.ds(sid * RPT, RPT)],
                    degp_hbm.at[cid, pl.ds(sid * RPT, RPT)])


@functools.partial(
    pl.kernel,
    out_type=jax.ShapeDtypeStruct((NC, NP, D_HID), jnp.float32),
    mesh=_sc_mesh(),
    scratch_types=[
        pltpu.VMEM((NCH, CH), jnp.int32),
        pltpu.VMEM((NCH, CH), jnp.int32),
        pltpu.VMEM((NBUF, CH, D_HID), jnp.float32),
        pltpu.VMEM_SHARED((NP, D_HID), jnp.float32),
        pltpu.SemaphoreType.DMA,
    ],
)
def _agg_kernel(g_hbm, srcm_hbm, dstm_hbm, acc_hbm,
                src_v, dst_v, rows_v, acc_sh, sem):
    cid = lax.axis_index("c")
    sid = lax.axis_index("s")
    wid = sid * NC + cid

    @pl.loop(0, 64)
    def _zrow(r):
        @pl.loop(0, D_HID // 16)
        def _zcol(c):
            rows_v[0, r, pl.ds(c * 16, 16)] = jnp.zeros((16,), jnp.float32)

    pltpu.sync_copy(srcm_hbm.at[wid], src_v)
    pltpu.sync_copy(dstm_hbm.at[wid], dst_v)

    @pl.loop(0, RPT // 64)
    def _zinit(k):
        pltpu.sync_copy(rows_v.at[0, pl.ds(0, 64)],
                        acc_sh.at[pl.ds(sid * RPT + k * 64, 64)])

    plsc.subcore_barrier()

    for j in range(NBUF - 1):
        pltpu.async_copy(g_hbm.at[src_v.at[j]], rows_v.at[j], sem)

    @pl.loop(0, NCH)
    def _edges(j):
        b = lax.rem(j, NBUF)
        pltpu.make_async_copy(g_hbm.at[src_v.at[j]], rows_v.at[b], sem).wait()

        @pl.when(j + NBUF - 1 < NCH)
        def _next():
            pltpu.async_copy(g_hbm.at[src_v.at[j + NBUF - 1]],
                             rows_v.at[lax.rem(j + NBUF - 1, NBUF)], sem)

        pltpu.sync_copy(rows_v.at[b], acc_sh.at[dst_v.at[j]], add=True)

    plsc.subcore_barrier()
    pltpu.sync_copy(acc_sh.at[pl.ds(sid * RPT, RPT)],
                    acc_hbm.at[cid, pl.ds(sid * RPT, RPT)])


_BM = 1000
_GRID = NN // _BM


def _mmh_body(x_ref, w_ref, h_ref):
    h_ref[...] = jnp.dot(x_ref[...], w_ref[...],
                         preferred_element_type=jnp.float32)


def _mmh(x, W1):
    return pl.pallas_call(
        _mmh_body,
        grid=(_GRID,),
        in_specs=[
            pl.BlockSpec((_BM, 256), lambda i: (i, 0)),
            pl.BlockSpec((256, D_HID), lambda i: (0, 0)),
        ],
        out_specs=pl.BlockSpec((_BM, D_HID), lambda i: (i, 0)),
        out_shape=jax.ShapeDtypeStruct((NP, D_HID), jnp.float32),
    )(x, W1)


def _scale_body(h_ref, degp_ref, g_ref, dis_ref):
    deg = degp_ref[0] + degp_ref[1] + 1.0
    dis = lax.rsqrt(deg)
    g_ref[...] = h_ref[...] * dis
    dis_ref[...] = dis


def _scale(h, degp):
    return pl.pallas_call(
        _scale_body,
        grid=(_GRID,),
        in_specs=[
            pl.BlockSpec((_BM, D_HID), lambda i: (i, 0)),
            pl.BlockSpec((NC, _BM, 1), lambda i: (0, i, 0)),
        ],
        out_specs=[
            pl.BlockSpec((_BM, D_HID), lambda i: (i, 0)),
            pl.BlockSpec((_BM, 1), lambda i: (i, 0)),
        ],
        out_shape=[
            jax.ShapeDtypeStruct((NP, D_HID), jnp.float32),
            jax.ShapeDtypeStruct((NP, 1), jnp.float32),
        ],
    )(h, degp.reshape(NC, NP, 1))


def _mid_body(acc_ref, g_ref, dis_ref, b1_ref, g2_ref):
    agg = dis_ref[...] * (acc_ref[0] + acc_ref[1] + g_ref[...])
    h1 = jnp.maximum(agg + b1_ref[...], 0.0)
    g2_ref[...] = h1 * dis_ref[...]


def _mid(acc1, g1, dis, b1):
    return pl.pallas_call(
        _mid_body,
        grid=(_GRID,),
        in_specs=[
            pl.BlockSpec((NC, _BM, D_HID), lambda i: (0, i, 0)),
            pl.BlockSpec((_BM, D_HID), lambda i: (i, 0)),
            pl.BlockSpec((_BM, 1), lambda i: (i, 0)),
            pl.BlockSpec((1, D_HID), lambda i: (0, 0)),
        ],
        out_specs=pl.BlockSpec((_BM, D_HID), lambda i: (i, 0)),
        out_shape=jax.ShapeDtypeStruct((NP, D_HID), jnp.float32),
    )(acc1, g1, dis, b1)


def _fin_body(acc_ref, g_ref, dis_ref, w_ref, b2_ref, out_ref):
    agg = dis_ref[...] * (acc_ref[0] + acc_ref[1] + g_ref[...])
    out_ref[...] = jnp.dot(agg, w_ref[...],
                           preferred_element_type=jnp.float32) + b2_ref[...]


def _fin(acc2, g2, dis, W2, b2):
    return pl.pallas_call(
        _fin_body,
        grid=(_GRID,),
        in_specs=[
            pl.BlockSpec((NC, _BM, D_HID), lambda i: (0, i, 0)),
            pl.BlockSpec((_BM, D_HID), lambda i: (i, 0)),
            pl.BlockSpec((_BM, 1), lambda i: (i, 0)),
            pl.BlockSpec((D_HID, 256), lambda i: (0, 0)),
            pl.BlockSpec((1, 256), lambda i: (0, 0)),
        ],
        out_specs=pl.BlockSpec((_BM, 256), lambda i: (i, 0)),
        out_shape=jax.ShapeDtypeStruct((NN, 256), jnp.float32),
    )(acc2, g2, dis, W2, b2)


def kernel(x, edge_index, W1, b1, W2, b2):
    src = edge_index[0]
    dst = edge_index[1]
    pad_idx = NN + (jnp.arange(EP - EE, dtype=jnp.int32) % (NP - NN))
    srcm = jnp.concatenate([src, pad_idx]).reshape(NW, NCH, CH)
    dstm = jnp.concatenate([dst, pad_idx]).reshape(NW, NCH, CH)

    degp = _deg_kernel(dstm.reshape(NW, DNCH, DCH))
    h = _mmh(x, W1)
    g1, dis = _scale(h, degp)
    acc1 = _agg_kernel(g1, srcm, dstm)
    g2 = _mid(acc1, g1, dis, b1.reshape(1, D_HID))
    acc2 = _agg_kernel(g2, srcm, dstm)
    return _fin(acc2, g2, dis, W2, b2.reshape(1, 256))


# speedup vs baseline: 1.0291x; 1.0291x over previous
import functools

import jax
import jax.numpy as jnp
from jax import lax
from jax.experimental import pallas as pl
from jax.experimental.pallas import tpu as pltpu
from jax.experimental.pallas import tpu_sc as plsc

NN = 10000
NP = 10240
EE = 160000
EP = 163840
NC = 2
NS = 16
NW = NC * NS
EW = EP // NW
CH = 64
NCH = EW // CH
RPT = NP // NS
D_HID = 128
NBUF = 3


def _sc_mesh():
    return plsc.VectorSubcoreMesh(core_axis_name="c", subcore_axis_name="s",
                                  num_cores=NC, num_subcores=NS)


DCH = 128
DNCH = EW // DCH


@functools.partial(
    pl.kernel,
    out_type=jax.ShapeDtypeStruct((NC, NP), jnp.float32),
    mesh=_sc_mesh(),
    scratch_types=[
        pltpu.VMEM((DNCH, DCH), jnp.int32),
        pltpu.VMEM((DCH,), jnp.float32),
        pltpu.VMEM((RPT,), jnp.float32),
        pltpu.VMEM_SHARED((NP,), jnp.float32),
    ],
)
def _deg_kernel(dstm_hbm, degp_hbm, dst_v, ones_v, zz_v, deg_sh):
    cid = lax.axis_index("c")
    sid = lax.axis_index("s")
    wid = sid * NC + cid

    @pl.loop(0, DCH // 16)
    def _fill_ones(i):
        ones_v[pl.ds(i * 16, 16)] = jnp.ones((16,), jnp.float32)

    @pl.loop(0, RPT // 16)
    def _fill_zero(i):
        zz_v[pl.ds(i * 16, 16)] = jnp.zeros((16,), jnp.float32)

    pltpu.sync_copy(dstm_hbm.at[wid], dst_v)
    pltpu.sync_copy(zz_v, deg_sh.at[pl.ds(sid * RPT, RPT)])
    plsc.subcore_barrier()

    @pl.loop(0, DNCH)
    def _scatter(j):
        pltpu.sync_copy(ones_v, deg_sh.at[dst_v.at[j]], add=True)

    plsc.subcore_barrier()
    pltpu.sync_copy(deg_sh.at[pl.ds(sid * RPT, RPT)],
                    degp_hbm.at[cid, pl.ds(sid * RPT, RPT)])


@functools.partial(
    pl.kernel,
    out_type=jax.ShapeDtypeStruct((NC, NP, D_HID), jnp.float32),
    mesh=_sc_mesh(),
    scratch_types=[
        pltpu.VMEM((NCH, CH), jnp.int32),
        pltpu.VMEM((NCH, CH), jnp.int32),
        pltpu.VMEM((NBUF, CH, D_HID), jnp.float32),
        pltpu.VMEM_SHARED((NP, D_HID), jnp.float32),
        pltpu.SemaphoreType.DMA,
    ],
)
def _agg_kernel(g_hbm, srcm_hbm, dstm_hbm, acc_hbm,
                src_v, dst_v, rows_v, acc_sh, sem):
    cid = lax.axis_index("c")
    sid = lax.axis_index("s")
    wid = sid * NC + cid

    @pl.loop(0, 64)
    def _zrow(r):
        @pl.loop(0, D_HID // 16)
        def _zcol(c):
            rows_v[0, r, pl.ds(c * 16, 16)] = jnp.zeros((16,), jnp.float32)

    pltpu.sync_copy(srcm_hbm.at[wid], src_v)
    pltpu.sync_copy(dstm_hbm.at[wid], dst_v)

    @pl.loop(0, RPT // 64)
    def _zinit(k):
        pltpu.sync_copy(rows_v.at[0, pl.ds(0, 64)],
                        acc_sh.at[pl.ds(sid * RPT + k * 64, 64)])

    plsc.subcore_barrier()

    for j in range(NBUF - 1):
        pltpu.async_copy(g_hbm.at[src_v.at[j]], rows_v.at[j], sem)

    @pl.loop(0, NCH)
    def _edges(j):
        b = lax.rem(j, NBUF)
        pltpu.make_async_copy(g_hbm.at[src_v.at[j]], rows_v.at[b], sem).wait()

        @pl.when(j + NBUF - 1 < NCH)
        def _next():
            pltpu.async_copy(g_hbm.at[src_v.at[j + NBUF - 1]],
                             rows_v.at[lax.rem(j + NBUF - 1, NBUF)], sem)

        pltpu.sync_copy(rows_v.at[b], acc_sh.at[dst_v.at[j]], add=True)

    plsc.subcore_barrier()
    pltpu.sync_copy(acc_sh.at[pl.ds(sid * RPT, RPT)],
                    acc_hbm.at[cid, pl.ds(sid * RPT, RPT)])


_BM = 1000
_GRID = NN // _BM


def _mm1_body(x_ref, w_ref, degp_ref, g_ref, dis_ref):
    deg = degp_ref[0] + degp_ref[1] + 1.0
    dis = lax.rsqrt(deg)
    h = jnp.dot(x_ref[...], w_ref[...], preferred_element_type=jnp.float32)
    g_ref[...] = h * dis
    dis_ref[...] = dis


def _mm1(x, W1, degp):
    return pl.pallas_call(
        _mm1_body,
        grid=(_GRID,),
        in_specs=[
            pl.BlockSpec((_BM, 256), lambda i: (i, 0)),
            pl.BlockSpec((256, D_HID), lambda i: (0, 0)),
            pl.BlockSpec((NC, _BM, 1), lambda i: (0, i, 0)),
        ],
        out_specs=[
            pl.BlockSpec((_BM, D_HID), lambda i: (i, 0)),
            pl.BlockSpec((_BM, 1), lambda i: (i, 0)),
        ],
        out_shape=[
            jax.ShapeDtypeStruct((NP, D_HID), jnp.float32),
            jax.ShapeDtypeStruct((NP, 1), jnp.float32),
        ],
    )(x, W1, degp.reshape(NC, NP, 1))


def _mid_body(acc_ref, g_ref, dis_ref, b1_ref, g2_ref):
    agg = dis_ref[...] * (acc_ref[0] + acc_ref[1] + g_ref[...])
    h1 = jnp.maximum(agg + b1_ref[...], 0.0)
    g2_ref[...] = h1 * dis_ref[...]


def _mid(acc1, g1, dis, b1):
    return pl.pallas_call(
        _mid_body,
        grid=(_GRID,),
        in_specs=[
            pl.BlockSpec((NC, _BM, D_HID), lambda i: (0, i, 0)),
            pl.BlockSpec((_BM, D_HID), lambda i: (i, 0)),
            pl.BlockSpec((_BM, 1), lambda i: (i, 0)),
            pl.BlockSpec((1, D_HID), lambda i: (0, 0)),
        ],
        out_specs=pl.BlockSpec((_BM, D_HID), lambda i: (i, 0)),
        out_shape=jax.ShapeDtypeStruct((NP, D_HID), jnp.float32),
    )(acc1, g1, dis, b1)


def _fin_body(acc_ref, g_ref, dis_ref, w_ref, b2_ref, out_ref):
    agg = dis_ref[...] * (acc_ref[0] + acc_ref[1] + g_ref[...])
    out_ref[...] = jnp.dot(agg, w_ref[...],
                           preferred_element_type=jnp.float32) + b2_ref[...]


def _fin(acc2, g2, dis, W2, b2):
    return pl.pallas_call(
        _fin_body,
        grid=(_GRID,),
        in_specs=[
            pl.BlockSpec((NC, _BM, D_HID), lambda i: (0, i, 0)),
            pl.BlockSpec((_BM, D_HID), lambda i: (i, 0)),
            pl.BlockSpec((_BM, 1), lambda i: (i, 0)),
            pl.BlockSpec((D_HID, 256), lambda i: (0, 0)),
            pl.BlockSpec((1, 256), lambda i: (0, 0)),
        ],
        out_specs=pl.BlockSpec((_BM, 256), lambda i: (i, 0)),
        out_shape=jax.ShapeDtypeStruct((NN, 256), jnp.float32),
    )(acc2, g2, dis, W2, b2)


def kernel(x, edge_index, W1, b1, W2, b2):
    src = edge_index[0]
    dst = edge_index[1]
    pad_idx = NN + (jnp.arange(EP - EE, dtype=jnp.int32) % (NP - NN))
    srcm = jnp.concatenate([src, pad_idx]).reshape(NW, NCH, CH)
    dstm = jnp.concatenate([dst, pad_idx]).reshape(NW, NCH, CH)

    degp = _deg_kernel(dstm.reshape(NW, DNCH, DCH))
    g1, dis = _mm1(x, W1, degp)
    acc1 = _agg_kernel(g1, srcm, dstm)
    g2 = _mid(acc1, g1, dis, b1.reshape(1, D_HID))
    acc2 = _agg_kernel(g2, srcm, dstm)
    return _fin(acc2, g2, dis, W2, b2.reshape(1, 256))


# speedup vs baseline: 1.0568x; 1.0269x over previous
import functools

import jax
import jax.numpy as jnp
from jax import lax
from jax.experimental import pallas as pl
from jax.experimental.pallas import tpu as pltpu
from jax.experimental.pallas import tpu_sc as plsc

NN = 10000
NP = 10240
EE = 160000
EP = 163840
NC = 2
NS = 16
NW = NC * NS
EW = EP // NW
CH = 64
NCH = EW // CH
RPT = NP // NS
D_HID = 128
NBUF = 3


def _sc_mesh():
    return plsc.VectorSubcoreMesh(core_axis_name="c", subcore_axis_name="s",
                                  num_cores=NC, num_subcores=NS)


DCH = 128
DNCH = EW // DCH


@functools.partial(
    pl.kernel,
    out_type=jax.ShapeDtypeStruct((NC, NP), jnp.float32),
    mesh=_sc_mesh(),
    scratch_types=[
        pltpu.VMEM((DNCH, DCH), jnp.int32),
        pltpu.VMEM((DCH,), jnp.float32),
        pltpu.VMEM((RPT,), jnp.float32),
        pltpu.VMEM_SHARED((NP,), jnp.float32),
    ],
)
def _deg_kernel(dstm_hbm, degp_hbm, dst_v, ones_v, zz_v, deg_sh):
    cid = lax.axis_index("c")
    sid = lax.axis_index("s")
    wid = sid * NC + cid

    @pl.loop(0, DCH // 16)
    def _fill_ones(i):
        ones_v[pl.ds(i * 16, 16)] = jnp.ones((16,), jnp.float32)

    @pl.loop(0, RPT // 16)
    def _fill_zero(i):
        zz_v[pl.ds(i * 16, 16)] = jnp.zeros((16,), jnp.float32)

    pltpu.sync_copy(dstm_hbm.at[wid], dst_v)
    pltpu.sync_copy(zz_v, deg_sh.at[pl.ds(sid * RPT, RPT)])
    plsc.subcore_barrier()

    @pl.loop(0, DNCH)
    def _scatter(j):
        pltpu.sync_copy(ones_v, deg_sh.at[dst_v.at[j]], add=True)

    plsc.subcore_barrier()
    pltpu.sync_copy(deg_sh.at[pl.ds(sid * RPT, RPT)],
                    degp_hbm.at[cid, pl.ds(sid * RPT, RPT)])


@functools.partial(
    pl.kernel,
    out_type=jax.ShapeDtypeStruct((NC, NP, D_HID), jnp.float32),
    mesh=_sc_mesh(),
    scratch_types=[
        pltpu.VMEM((NCH, CH), jnp.int32),
        pltpu.VMEM((NCH, CH), jnp.int32),
        pltpu.VMEM((NBUF, CH, D_HID), jnp.float32),
        pltpu.VMEM_SHARED((NP, D_HID), jnp.float32),
        pltpu.SemaphoreType.DMA,
    ],
)
def _agg_kernel(g_hbm, srcm_hbm, dstm_hbm, acc_hbm,
                src_v, dst_v, rows_v, acc_sh, sem):
    cid = lax.axis_index("c")
    sid = lax.axis_index("s")
    wid = sid * NC + cid

    @pl.loop(0, 64)
    def _zrow(r):
        @pl.loop(0, D_HID // 16)
        def _zcol(c):
            rows_v[0, r, pl.ds(c * 16, 16)] = jnp.zeros((16,), jnp.float32)

    pltpu.sync_copy(srcm_hbm.at[wid], src_v)
    pltpu.sync_copy(dstm_hbm.at[wid], dst_v)

    @pl.loop(0, RPT // 64)
    def _zinit(k):
        pltpu.sync_copy(rows_v.at[0, pl.ds(0, 64)],
                        acc_sh.at[pl.ds(sid * RPT + k * 64, 64)])

    plsc.subcore_barrier()

    for j in range(NBUF - 1):
        pltpu.async_copy(g_hbm.at[src_v.at[j]], rows_v.at[j], sem)

    @pl.loop(0, NCH)
    def _edges(j):
        b = lax.rem(j, NBUF)
        pltpu.make_async_copy(g_hbm.at[src_v.at[j]], rows_v.at[b], sem).wait()

        @pl.when(j + NBUF - 1 < NCH)
        def _next():
            pltpu.async_copy(g_hbm.at[src_v.at[j + NBUF - 1]],
                             rows_v.at[lax.rem(j + NBUF - 1, NBUF)], sem)

        pltpu.sync_copy(rows_v.at[b], acc_sh.at[dst_v.at[j]], add=True)

    plsc.subcore_barrier()
    pltpu.sync_copy(acc_sh.at[pl.ds(sid * RPT, RPT)],
                    acc_hbm.at[cid, pl.ds(sid * RPT, RPT)])


_BM = 2000
_GRID = NN // _BM


def _mm1_body(x_ref, w_ref, degp_ref, g_ref, dis_ref):
    deg = degp_ref[0] + degp_ref[1] + 1.0
    dis = lax.rsqrt(deg)
    h = jnp.dot(x_ref[...], w_ref[...], preferred_element_type=jnp.float32)
    g_ref[...] = h * dis
    dis_ref[...] = dis


def _mm1(x, W1, degp):
    return pl.pallas_call(
        _mm1_body,
        grid=(_GRID,),
        in_specs=[
            pl.BlockSpec((_BM, 256), lambda i: (i, 0)),
            pl.BlockSpec((256, D_HID), lambda i: (0, 0)),
            pl.BlockSpec((NC, _BM, 1), lambda i: (0, i, 0)),
        ],
        out_specs=[
            pl.BlockSpec((_BM, D_HID), lambda i: (i, 0)),
            pl.BlockSpec((_BM, 1), lambda i: (i, 0)),
        ],
        out_shape=[
            jax.ShapeDtypeStruct((NP, D_HID), jnp.float32),
            jax.ShapeDtypeStruct((NP, 1), jnp.float32),
        ],
    )(x, W1, degp.reshape(NC, NP, 1))


def _mid_body(acc_ref, g_ref, dis_ref, b1_ref, g2_ref):
    agg = dis_ref[...] * (acc_ref[0] + acc_ref[1] + g_ref[...])
    h1 = jnp.maximum(agg + b1_ref[...], 0.0)
    g2_ref[...] = h1 * dis_ref[...]


def _mid(acc1, g1, dis, b1):
    return pl.pallas_call(
        _mid_body,
        grid=(_GRID,),
        in_specs=[
            pl.BlockSpec((NC, _BM, D_HID), lambda i: (0, i, 0)),
            pl.BlockSpec((_BM, D_HID), lambda i: (i, 0)),
            pl.BlockSpec((_BM, 1), lambda i: (i, 0)),
            pl.BlockSpec((1, D_HID), lambda i: (0, 0)),
        ],
        out_specs=pl.BlockSpec((_BM, D_HID), lambda i: (i, 0)),
        out_shape=jax.ShapeDtypeStruct((NP, D_HID), jnp.float32),
    )(acc1, g1, dis, b1)


def _fin_body(acc_ref, g_ref, dis_ref, w_ref, b2_ref, out_ref):
    agg = dis_ref[...] * (acc_ref[0] + acc_ref[1] + g_ref[...])
    out_ref[...] = jnp.dot(agg, w_ref[...],
                           preferred_element_type=jnp.float32) + b2_ref[...]


def _fin(acc2, g2, dis, W2, b2):
    return pl.pallas_call(
        _fin_body,
        grid=(_GRID,),
        in_specs=[
            pl.BlockSpec((NC, _BM, D_HID), lambda i: (0, i, 0)),
            pl.BlockSpec((_BM, D_HID), lambda i: (i, 0)),
            pl.BlockSpec((_BM, 1), lambda i: (i, 0)),
            pl.BlockSpec((D_HID, 256), lambda i: (0, 0)),
            pl.BlockSpec((1, 256), lambda i: (0, 0)),
        ],
        out_specs=pl.BlockSpec((_BM, 256), lambda i: (i, 0)),
        out_shape=jax.ShapeDtypeStruct((NN, 256), jnp.float32),
    )(acc2, g2, dis, W2, b2)


def kernel(x, edge_index, W1, b1, W2, b2):
    src = edge_index[0]
    dst = edge_index[1]
    pad_idx = NN + (jnp.arange(EP - EE, dtype=jnp.int32) % (NP - NN))
    srcm = jnp.concatenate([src, pad_idx]).reshape(NW, NCH, CH)
    dstm = jnp.concatenate([dst, pad_idx]).reshape(NW, NCH, CH)

    degp = _deg_kernel(dstm.reshape(NW, DNCH, DCH))
    g1, dis = _mm1(x, W1, degp)
    acc1 = _agg_kernel(g1, srcm, dstm)
    g2 = _mid(acc1, g1, dis, b1.reshape(1, D_HID))
    acc2 = _agg_kernel(g2, srcm, dstm)
    return _fin(acc2, g2, dis, W2, b2.reshape(1, 256))


# speedup vs baseline: 1.0746x; 1.0168x over previous
import functools

import jax
import jax.numpy as jnp
from jax import lax
from jax.experimental import pallas as pl
from jax.experimental.pallas import tpu as pltpu
from jax.experimental.pallas import tpu_sc as plsc

NN = 10000
NP = 10240
EE = 160000
EP = 163840
NC = 2
NS = 16
NW = NC * NS
EW = EP // NW
CH = 64
NCH = EW // CH
RPT = NP // NS
D_HID = 128
NBUF = 3


def _sc_mesh():
    return plsc.VectorSubcoreMesh(core_axis_name="c", subcore_axis_name="s",
                                  num_cores=NC, num_subcores=NS)


DCH = 128
DNCH = EW // DCH


@functools.partial(
    pl.kernel,
    out_type=jax.ShapeDtypeStruct((NC, NP), jnp.float32),
    mesh=_sc_mesh(),
    scratch_types=[
        pltpu.VMEM((DNCH, DCH), jnp.int32),
        pltpu.VMEM((DCH,), jnp.float32),
        pltpu.VMEM((RPT,), jnp.float32),
        pltpu.VMEM_SHARED((NP,), jnp.float32),
    ],
)
def _deg_kernel(dstm_hbm, degp_hbm, dst_v, ones_v, zz_v, deg_sh):
    cid = lax.axis_index("c")
    sid = lax.axis_index("s")
    wid = sid * NC + cid

    @pl.loop(0, DCH // 16)
    def _fill_ones(i):
        ones_v[pl.ds(i * 16, 16)] = jnp.ones((16,), jnp.float32)

    @pl.loop(0, RPT // 16)
    def _fill_zero(i):
        zz_v[pl.ds(i * 16, 16)] = jnp.zeros((16,), jnp.float32)

    pltpu.sync_copy(dstm_hbm.at[wid], dst_v)
    pltpu.sync_copy(zz_v, deg_sh.at[pl.ds(sid * RPT, RPT)])
    plsc.subcore_barrier()

    @pl.loop(0, DNCH)
    def _scatter(j):
        pltpu.sync_copy(ones_v, deg_sh.at[dst_v.at[j]], add=True)

    plsc.subcore_barrier()
    pltpu.sync_copy(deg_sh.at[pl.ds(sid * RPT, RPT)],
                    degp_hbm.at[cid, pl.ds(sid * RPT, RPT)])


@functools.partial(
    pl.kernel,
    out_type=jax.ShapeDtypeStruct((NC, NP, D_HID), jnp.float32),
    mesh=_sc_mesh(),
    scratch_types=[
        pltpu.VMEM((NCH, CH), jnp.int32),
        pltpu.VMEM((NCH, CH), jnp.int32),
        pltpu.VMEM((NBUF, CH, D_HID), jnp.float32),
        pltpu.VMEM_SHARED((NP, D_HID), jnp.float32),
        pltpu.SemaphoreType.DMA,
    ],
)
def _agg_kernel(g_hbm, srcm_hbm, dstm_hbm, acc_hbm,
                src_v, dst_v, rows_v, acc_sh, sem):
    cid = lax.axis_index("c")
    sid = lax.axis_index("s")
    wid = sid * NC + cid

    @pl.loop(0, 64)
    def _zrow(r):
        @pl.loop(0, D_HID // 16)
        def _zcol(c):
            rows_v[0, r, pl.ds(c * 16, 16)] = jnp.zeros((16,), jnp.float32)

    pltpu.sync_copy(srcm_hbm.at[wid], src_v)
    pltpu.sync_copy(dstm_hbm.at[wid], dst_v)

    @pl.loop(0, RPT // 64)
    def _zinit(k):
        pltpu.sync_copy(rows_v.at[0, pl.ds(0, 64)],
                        acc_sh.at[pl.ds(sid * RPT + k * 64, 64)])

    plsc.subcore_barrier()

    for j in range(NBUF - 1):
        pltpu.async_copy(g_hbm.at[src_v.at[j]], rows_v.at[j], sem)

    @pl.loop(0, NCH)
    def _edges(j):
        b = lax.rem(j, NBUF)
        pltpu.make_async_copy(g_hbm.at[src_v.at[j]], rows_v.at[b], sem).wait()

        @pl.when(j + NBUF - 1 < NCH)
        def _next():
            pltpu.async_copy(g_hbm.at[src_v.at[j + NBUF - 1]],
                             rows_v.at[lax.rem(j + NBUF - 1, NBUF)], sem)

        pltpu.sync_copy(rows_v.at[b], acc_sh.at[dst_v.at[j]], add=True)

    plsc.subcore_barrier()
    pltpu.sync_copy(acc_sh.at[pl.ds(sid * RPT, RPT)],
                    acc_hbm.at[cid, pl.ds(sid * RPT, RPT)])


_BM = 5000
_GRID = NN // _BM


def _mm1_body(x_ref, w_ref, degp_ref, g_ref, dis_ref):
    deg = degp_ref[0] + degp_ref[1] + 1.0
    dis = lax.rsqrt(deg)
    h = jnp.dot(x_ref[...], w_ref[...], preferred_element_type=jnp.float32)
    g_ref[...] = h * dis
    dis_ref[...] = dis


def _mm1(x, W1, degp):
    return pl.pallas_call(
        _mm1_body,
        grid=(_GRID,),
        in_specs=[
            pl.BlockSpec((_BM, 256), lambda i: (i, 0)),
            pl.BlockSpec((256, D_HID), lambda i: (0, 0)),
            pl.BlockSpec((NC, _BM, 1), lambda i: (0, i, 0)),
        ],
        out_specs=[
            pl.BlockSpec((_BM, D_HID), lambda i: (i, 0)),
            pl.BlockSpec((_BM, 1), lambda i: (i, 0)),
        ],
        out_shape=[
            jax.ShapeDtypeStruct((NP, D_HID), jnp.float32),
            jax.ShapeDtypeStruct((NP, 1), jnp.float32),
        ],
    )(x, W1, degp.reshape(NC, NP, 1))


def _mid_body(acc_ref, g_ref, dis_ref, b1_ref, g2_ref):
    agg = dis_ref[...] * (acc_ref[0] + acc_ref[1] + g_ref[...])
    h1 = jnp.maximum(agg + b1_ref[...], 0.0)
    g2_ref[...] = h1 * dis_ref[...]


def _mid(acc1, g1, dis, b1):
    return pl.pallas_call(
        _mid_body,
        grid=(_GRID,),
        in_specs=[
            pl.BlockSpec((NC, _BM, D_HID), lambda i: (0, i, 0)),
            pl.BlockSpec((_BM, D_HID), lambda i: (i, 0)),
            pl.BlockSpec((_BM, 1), lambda i: (i, 0)),
            pl.BlockSpec((1, D_HID), lambda i: (0, 0)),
        ],
        out_specs=pl.BlockSpec((_BM, D_HID), lambda i: (i, 0)),
        out_shape=jax.ShapeDtypeStruct((NP, D_HID), jnp.float32),
    )(acc1, g1, dis, b1)


def _fin_body(acc_ref, g_ref, dis_ref, w_ref, b2_ref, out_ref):
    agg = dis_ref[...] * (acc_ref[0] + acc_ref[1] + g_ref[...])
    out_ref[...] = jnp.dot(agg, w_ref[...],
                           preferred_element_type=jnp.float32) + b2_ref[...]


def _fin(acc2, g2, dis, W2, b2):
    return pl.pallas_call(
        _fin_body,
        grid=(_GRID,),
        in_specs=[
            pl.BlockSpec((NC, _BM, D_HID), lambda i: (0, i, 0)),
            pl.BlockSpec((_BM, D_HID), lambda i: (i, 0)),
            pl.BlockSpec((_BM, 1), lambda i: (i, 0)),
            pl.BlockSpec((D_HID, 256), lambda i: (0, 0)),
            pl.BlockSpec((1, 256), lambda i: (0, 0)),
        ],
        out_specs=pl.BlockSpec((_BM, 256), lambda i: (i, 0)),
        out_shape=jax.ShapeDtypeStruct((NN, 256), jnp.float32),
    )(acc2, g2, dis, W2, b2)


def kernel(x, edge_index, W1, b1, W2, b2):
    src = edge_index[0]
    dst = edge_index[1]
    pad_idx = NN + (jnp.arange(EP - EE, dtype=jnp.int32) % (NP - NN))
    srcm = jnp.concatenate([src, pad_idx]).reshape(NW, NCH, CH)
    dstm = jnp.concatenate([dst, pad_idx]).reshape(NW, NCH, CH)

    degp = _deg_kernel(dstm.reshape(NW, DNCH, DCH))
    g1, dis = _mm1(x, W1, degp)
    acc1 = _agg_kernel(g1, srcm, dstm)
    g2 = _mid(acc1, g1, dis, b1.reshape(1, D_HID))
    acc2 = _agg_kernel(g2, srcm, dstm)
    return _fin(acc2, g2, dis, W2, b2.reshape(1, 256))
